# baseline (speedup 1.0000x reference)
import functools

import jax
import jax.numpy as jnp
from jax import lax
from jax.experimental import pallas as pl
from jax.experimental.pallas import tpu as pltpu
from jax.experimental.pallas import tpu_sc as plsc

_NSLOT = 16
_GROUP = 128


def _make_tiled_gather(num_users, embed_dim, batch):
    info = plsc.get_sparse_core_info()
    nc, ns = info.num_cores, info.num_subcores
    nw = nc * ns
    assert batch % (_GROUP * nw) == 0
    b_per_w = batch // nw
    n_groups = b_per_w // _GROUP
    rounds_per_group = _GROUP // _NSLOT
    mesh = plsc.VectorSubcoreMesh(core_axis_name="c", subcore_axis_name="s")

    @functools.partial(
        pl.kernel,
        mesh=mesh,
        compiler_params=pltpu.CompilerParams(needs_layout_passes=False),
        out_type=jax.ShapeDtypeStruct((embed_dim, batch), jnp.float32),
        scratch_types=[
            pltpu.VMEM((b_per_w,), jnp.int32),
            pltpu.VMEM((_NSLOT // 2, embed_dim, 128), jnp.float32),
            pltpu.VMEM_SHARED((ns, _NSLOT // 2, embed_dim, 128), jnp.float32),
            pltpu.VMEM((2, embed_dim, 128), jnp.float32),
            [pltpu.SemaphoreType.DMA] * (_NSLOT // 2),
            [pltpu.SemaphoreType.DMA] * (_NSLOT // 2),
            pltpu.SemaphoreType.DMA,
        ],
    )
    def tiled_gather(uidx_hbm, utab_hbm, uout_hbm,
                     uidx_v, slots_v, sp_v, obuf_v, slot_sems, sp_sems, wsem):
        sid = lax.axis_index("s")
        wid = sid * nc + lax.axis_index("c")
        base = wid * b_per_w
        pltpu.sync_copy(uidx_hbm.at[pl.ds(base, b_per_w)], uidx_v)

        iota16 = lax.iota(jnp.int32, 16)

        def fetch(tab_hbm, i, p):
            aligned = pl.multiple_of((i >> 7) << 7, 128)
            slot = p // 2
            if p % 2 == 0:
                pltpu.async_copy(
                    tab_hbm.at[:, pl.ds(aligned, 128)],
                    slots_v.at[slot],
                    slot_sems[slot],
                )
            else:
                pltpu.async_copy(
                    tab_hbm.at[:, pl.ds(aligned, 128)],
                    sp_v.at[sid, slot],
                    sp_sems[slot],
                )

        def wait_slot(tab_hbm, p):
            slot = p // 2
            if p % 2 == 0:
                pltpu.make_async_copy(
                    tab_hbm.at[:, pl.ds(0, 128)],
                    slots_v.at[slot],
                    slot_sems[slot],
                ).wait()
            else:
                pltpu.make_async_copy(
                    tab_hbm.at[:, pl.ds(0, 128)],
                    sp_v.at[sid, slot],
                    sp_sems[slot],
                ).wait()

        def extract(jl, i, p, buf):
            slot = p // 2
            if p % 2 == 0:
                jc = jnp.full((16,), slot, jnp.int32)
                bufv = jnp.full((16,), buf, jnp.int32)
                dstc = lax.broadcast_in_dim(jl, (16,), ())
                colv = lax.broadcast_in_dim(i & 127, (16,), ())
                for h in range(embed_dim // 16):
                    rows = iota16 + (16 * h)
                    vals = plsc.load_gather(slots_v, [jc, rows, colv])
                    plsc.store_scatter(obuf_v, [bufv, rows, dstc], vals)
            else:
                pltpu.sync_copy(
                    sp_v.at[sid, slot, :, i & 127],
                    obuf_v.at[buf, :, jl],
                )

        def run_table(tab_hbm, idx_v, out_hbm):
            n_rounds = b_per_w // _NSLOT
            ivec0 = idx_v[pl.ds(0, _NSLOT)]
            for p in range(_NSLOT):
                fetch(tab_hbm, ivec0[p], p)
            for g in range(n_groups):
                buf = g % 2
                g0 = g * _GROUP

                def round_body(rl, carry, _g=g, _g0=g0, _buf=buf):
                    r = _g * rounds_per_group + rl
                    ivec = idx_v[pl.ds(r * _NSLOT, _NSLOT)]
                    has_next = r < n_rounds - 1
                    for p in range(_NSLOT):
                        wait_slot(tab_hbm, p)
                        extract(rl * _NSLOT + p, ivec[p], p, _buf)

                    @pl.when(has_next)
                    def _():
                        nvec = idx_v[pl.ds((r + 1) * _NSLOT, _NSLOT)]
                        for p in range(_NSLOT):
                            fetch(tab_hbm, nvec[p], p)

                    return carry

                lax.fori_loop(0, rounds_per_group, round_body, 0)
                pltpu.async_copy(
                    obuf_v.at[buf],
                    out_hbm.at[:, pl.ds(base + g0, _GROUP)],
                    wsem,
                )
                if g >= 1:
                    pltpu.make_async_copy(
                        obuf_v.at[buf],
                        out_hbm.at[:, pl.ds(base + g0, _GROUP)],
                        wsem,
                    ).wait()
            pltpu.make_async_copy(
                obuf_v.at[0],
                out_hbm.at[:, pl.ds(base, _GROUP)],
                wsem,
            ).wait()

        run_table(utab_hbm, uidx_v, uout_hbm)

    return tiled_gather


def _make_row_gather(num_rows, embed_dim, batch):
    info = plsc.get_sparse_core_info()
    nc, ns = info.num_cores, info.num_subcores
    nw = nc * ns
    b_per_w = batch // nw
    mesh = plsc.VectorSubcoreMesh(core_axis_name="c", subcore_axis_name="s")

    @functools.partial(
        pl.kernel,
        mesh=mesh,
        compiler_params=pltpu.CompilerParams(use_tc_tiling_on_sc=False),
        out_type=jax.ShapeDtypeStruct((batch, embed_dim), jnp.float32),
        scratch_types=[
            pltpu.VMEM((b_per_w,), jnp.int32),
            pltpu.VMEM((b_per_w, embed_dim), jnp.float32),
            pltpu.SemaphoreType.DMA,
        ],
    )
    def row_gather(idx_hbm, tab_hbm, out_hbm, idx_v, rows_v, sem):
        wid = lax.axis_index("s") * nc + lax.axis_index("c")
        base = wid * b_per_w
        pltpu.sync_copy(idx_hbm.at[pl.ds(base, b_per_w)], idx_v)
        pltpu.async_copy(tab_hbm.at[idx_v], rows_v, sem).wait()
        pltpu.sync_copy(rows_v, out_hbm.at[pl.ds(base, b_per_w)])

    return row_gather


def kernel(user_input, item_input, user_table, item_table):
    batch = user_input.shape[0]
    num_users, embed_dim = user_table.shape
    num_items, _ = item_table.shape
    user_fn = _make_tiled_gather(num_users, embed_dim, batch)
    item_fn = _make_row_gather(num_items, embed_dim, batch)
    out_u_t = user_fn(user_input.astype(jnp.int32), user_table.T)
    out_i = item_fn(item_input.astype(jnp.int32), item_table)
    return (out_u_t.T, out_i)


# speedup vs baseline: 1.1309x; 1.1309x over previous
import functools

import jax
import jax.numpy as jnp
from jax import lax
from jax.experimental import pallas as pl
from jax.experimental.pallas import tpu as pltpu
from jax.experimental.pallas import tpu_sc as plsc

_NSLOT = 16
_GROUP = 128


def _make_tiled_gather(num_users, embed_dim, batch):
    info = plsc.get_sparse_core_info()
    nc, ns = info.num_cores, info.num_subcores
    nw = nc * ns
    assert batch % (_GROUP * nw) == 0
    b_per_w = batch // nw
    n_groups = b_per_w // _GROUP
    rounds_per_group = _GROUP // _NSLOT
    mesh = plsc.VectorSubcoreMesh(core_axis_name="c", subcore_axis_name="s")

    @functools.partial(
        pl.kernel,
        mesh=mesh,
        compiler_params=pltpu.CompilerParams(needs_layout_passes=False),
        out_type=jax.ShapeDtypeStruct((embed_dim, batch), jnp.float32),
        scratch_types=[
            pltpu.VMEM((b_per_w,), jnp.int32),
            pltpu.VMEM((_NSLOT, embed_dim, 128), jnp.float32),
            pltpu.VMEM((2, embed_dim, 128), jnp.float32),
            [pltpu.SemaphoreType.DMA] * _NSLOT,
            pltpu.SemaphoreType.DMA,
        ],
    )
    def tiled_gather(uidx_hbm, utab_hbm, uout_hbm,
                     uidx_v, slots_v, obuf_v, slot_sems, wsem):
        wid = lax.axis_index("s") * nc + lax.axis_index("c")
        base = wid * b_per_w
        pltpu.sync_copy(uidx_hbm.at[pl.ds(base, b_per_w)], uidx_v)

        iota16 = lax.iota(jnp.int32, 16)

        def fetch(tab_hbm, i, slot):
            aligned = pl.multiple_of((i >> 7) << 7, 128)
            pltpu.async_copy(
                tab_hbm.at[:, pl.ds(aligned, 128)],
                slots_v.at[slot],
                slot_sems[slot],
            )

        def wait_slot(tab_hbm, slot):
            pltpu.make_async_copy(
                tab_hbm.at[:, pl.ds(0, 128)],
                slots_v.at[slot],
                slot_sems[slot],
            ).wait()

        def extract(jl, i, slot, buf):
            jc = jnp.full((16,), slot, jnp.int32)
            bufv = jnp.full((16,), buf, jnp.int32)
            dstc = lax.broadcast_in_dim(jl, (16,), ())
            colv = lax.broadcast_in_dim(i & 127, (16,), ())
            for h in range(embed_dim // 16):
                rows = iota16 + (16 * h)
                vals = plsc.load_gather(slots_v, [jc, rows, colv])
                plsc.store_scatter(obuf_v, [bufv, rows, dstc], vals)

        def run_table(tab_hbm, idx_v, out_hbm):
            n_rounds = b_per_w // _NSLOT
            ivec0 = idx_v[pl.ds(0, _NSLOT)]
            for p in range(_NSLOT):
                fetch(tab_hbm, ivec0[p], p)
            for g in range(n_groups):
                buf = g % 2
                g0 = g * _GROUP

                def round_body(rl, carry, _g=g, _g0=g0, _buf=buf):
                    r = _g * rounds_per_group + rl
                    ivec = idx_v[pl.ds(r * _NSLOT, _NSLOT)]
                    has_next = r < n_rounds - 1
                    for p in range(_NSLOT):
                        wait_slot(tab_hbm, p)
                        extract(rl * _NSLOT + p, ivec[p], p, _buf)

                    @pl.when(has_next)
                    def _():
                        nvec = idx_v[pl.ds((r + 1) * _NSLOT, _NSLOT)]
                        for p in range(_NSLOT):
                            fetch(tab_hbm, nvec[p], p)

                    return carry

                lax.fori_loop(0, rounds_per_group, round_body, 0)
                pltpu.async_copy(
                    obuf_v.at[buf],
                    out_hbm.at[:, pl.ds(base + g0, _GROUP)],
                    wsem,
                )
                if g >= 1:
                    pltpu.make_async_copy(
                        obuf_v.at[buf],
                        out_hbm.at[:, pl.ds(base + g0, _GROUP)],
                        wsem,
                    ).wait()
            pltpu.make_async_copy(
                obuf_v.at[0],
                out_hbm.at[:, pl.ds(base, _GROUP)],
                wsem,
            ).wait()

        run_table(utab_hbm, uidx_v, uout_hbm)

    return tiled_gather


def _make_row_gather(num_rows, embed_dim, batch):
    info = plsc.get_sparse_core_info()
    nc, ns = info.num_cores, info.num_subcores
    nw = nc * ns
    b_per_w = batch // nw
    mesh = plsc.VectorSubcoreMesh(core_axis_name="c", subcore_axis_name="s")

    @functools.partial(
        pl.kernel,
        mesh=mesh,
        compiler_params=pltpu.CompilerParams(use_tc_tiling_on_sc=False),
        out_type=jax.ShapeDtypeStruct((batch, embed_dim), jnp.float32),
        scratch_types=[
            pltpu.VMEM((b_per_w,), jnp.int32),
            pltpu.VMEM((b_per_w, embed_dim), jnp.float32),
            pltpu.SemaphoreType.DMA,
        ],
    )
    def row_gather(idx_hbm, tab_hbm, out_hbm, idx_v, rows_v, sem):
        wid = lax.axis_index("s") * nc + lax.axis_index("c")
        base = wid * b_per_w
        pltpu.sync_copy(idx_hbm.at[pl.ds(base, b_per_w)], idx_v)
        pltpu.async_copy(tab_hbm.at[idx_v], rows_v, sem).wait()
        pltpu.sync_copy(rows_v, out_hbm.at[pl.ds(base, b_per_w)])

    return row_gather


def kernel(user_input, item_input, user_table, item_table):
    batch = user_input.shape[0]
    num_users, embed_dim = user_table.shape
    num_items, _ = item_table.shape
    user_fn = _make_tiled_gather(num_users, embed_dim, batch)
    item_fn = _make_row_gather(num_items, embed_dim, batch)
    out_u_t = user_fn(user_input.astype(jnp.int32), user_table.T)
    out_i = item_fn(item_input.astype(jnp.int32), item_table)
    return (out_u_t.T, out_i)


# speedup vs baseline: 1.1327x; 1.0016x over previous
import functools

import jax
import jax.numpy as jnp
from jax import lax
from jax.experimental import pallas as pl
from jax.experimental.pallas import tpu as pltpu
from jax.experimental.pallas import tpu_sc as plsc

_NSLOT = 16
_GROUP = 128


def _make_tiled_gather(num_users, embed_dim, batch):
    info = plsc.get_sparse_core_info()
    nc, ns = info.num_cores, info.num_subcores
    nw = nc * ns
    assert batch % (_GROUP * nw) == 0
    b_per_w = batch // nw
    n_groups = b_per_w // _GROUP
    rounds_per_group = _GROUP // _NSLOT
    mesh = plsc.VectorSubcoreMesh(core_axis_name="c", subcore_axis_name="s")

    @functools.partial(
        pl.kernel,
        mesh=mesh,
        compiler_params=pltpu.CompilerParams(needs_layout_passes=False),
        out_type=jax.ShapeDtypeStruct((embed_dim, batch), jnp.float32),
        scratch_types=[
            pltpu.VMEM((b_per_w,), jnp.int32),
            pltpu.VMEM((_NSLOT, embed_dim, 128), jnp.float32),
            pltpu.VMEM((2, embed_dim, 128), jnp.float32),
            [pltpu.SemaphoreType.DMA] * _NSLOT,
            pltpu.SemaphoreType.DMA,
        ],
    )
    def tiled_gather(uidx_hbm, utab_hbm, uout_hbm,
                     uidx_v, slots_v, obuf_v, slot_sems, wsem):
        wid = lax.axis_index("s") * nc + lax.axis_index("c")
        base = wid * b_per_w
        pltpu.sync_copy(uidx_hbm.at[pl.ds(base, b_per_w)], uidx_v)

        iota16 = lax.iota(jnp.int32, 16)

        def fetch(tab_hbm, i, slot):
            aligned = pl.multiple_of((i >> 7) << 7, 128)
            pltpu.async_copy(
                tab_hbm.at[:, pl.ds(aligned, 128)],
                slots_v.at[slot],
                slot_sems[slot],
            )

        def wait_slot(tab_hbm, slot):
            pltpu.make_async_copy(
                tab_hbm.at[:, pl.ds(0, 128)],
                slots_v.at[slot],
                slot_sems[slot],
            ).wait()

        def extract(jl, i, slot, buf):
            jc = jnp.full((16,), slot, jnp.int32)
            bufv = jnp.full((16,), buf, jnp.int32)
            dstc = lax.broadcast_in_dim(jl, (16,), ())
            colv = lax.broadcast_in_dim(i & 127, (16,), ())
            for h in range(embed_dim // 16):
                rows = iota16 + (16 * h)
                vals = plsc.load_gather(slots_v, [jc, rows, colv])
                plsc.store_scatter(obuf_v, [bufv, rows, dstc], vals)

        def run_table(tab_hbm, idx_v, out_hbm):
            n_rounds = b_per_w // _NSLOT
            ivec0 = idx_v[pl.ds(0, _NSLOT)]
            for p in range(_NSLOT):
                fetch(tab_hbm, ivec0[p], p)
            for g in range(n_groups):
                buf = g % 2
                g0 = g * _GROUP

                def round_body(rl, carry, _g=g, _g0=g0, _buf=buf):
                    r = _g * rounds_per_group + rl
                    ivec = idx_v[pl.ds(r * _NSLOT, _NSLOT)]
                    has_next = r < n_rounds - 1
                    for p in range(_NSLOT):
                        wait_slot(tab_hbm, p)
                        extract(rl * _NSLOT + p, ivec[p], p, _buf)

                    @pl.when(has_next)
                    def _():
                        nvec = idx_v[pl.ds((r + 1) * _NSLOT, _NSLOT)]
                        for p in range(_NSLOT):
                            fetch(tab_hbm, nvec[p], p)

                    return carry

                lax.fori_loop(0, rounds_per_group, round_body, 0)
                pltpu.async_copy(
                    obuf_v.at[buf],
                    out_hbm.at[:, pl.ds(base + g0, _GROUP)],
                    wsem,
                )
                if g >= 1:
                    pltpu.make_async_copy(
                        obuf_v.at[buf],
                        out_hbm.at[:, pl.ds(base + g0, _GROUP)],
                        wsem,
                    ).wait()
            pltpu.make_async_copy(
                obuf_v.at[0],
                out_hbm.at[:, pl.ds(base, _GROUP)],
                wsem,
            ).wait()

        run_table(utab_hbm, uidx_v, uout_hbm)

    return tiled_gather


def _make_row_gather(num_rows, embed_dim, batch):
    info = plsc.get_sparse_core_info()
    nc, ns = info.num_cores, info.num_subcores
    nw = nc * ns
    b_per_w = batch // nw
    mesh = plsc.VectorSubcoreMesh(core_axis_name="c", subcore_axis_name="s")

    @functools.partial(
        pl.kernel,
        mesh=mesh,
        compiler_params=pltpu.CompilerParams(use_tc_tiling_on_sc=False),
        out_type=jax.ShapeDtypeStruct((batch, embed_dim), jnp.float32),
        scratch_types=[
            pltpu.VMEM((b_per_w,), jnp.int32),
            pltpu.VMEM((b_per_w, embed_dim), jnp.float32),
            pltpu.SemaphoreType.DMA,
        ],
    )
    def row_gather(idx_hbm, tab_hbm, out_hbm, idx_v, rows_v, sem):
        wid = lax.axis_index("s") * nc + lax.axis_index("c")
        base = wid * b_per_w
        pltpu.sync_copy(idx_hbm.at[pl.ds(base, b_per_w)], idx_v)
        pltpu.async_copy(tab_hbm.at[idx_v], rows_v, sem).wait()
        pltpu.sync_copy(rows_v, out_hbm.at[pl.ds(base, b_per_w)])

    return row_gather


def kernel(user_input, item_input, user_table, item_table):
    batch = user_input.shape[0]
    num_users, embed_dim = user_table.shape
    num_items, _ = item_table.shape
    user_fn = _make_tiled_gather(num_users, embed_dim, batch)
    item_fn = _make_row_gather(num_items, embed_dim, batch)
    out_i = item_fn(item_input.astype(jnp.int32), item_table)
    out_u_t = user_fn(user_input.astype(jnp.int32), user_table.T)
    return (out_u_t.T, out_i)
